# baseline (speedup 1.0000x reference)
import functools

import jax
import jax.numpy as jnp
from jax.experimental import pallas as pl

N = 11616
NHID = 128
ROW_BLK = 352
N_BLKS = N // ROW_BLK
GAMMA = 2.0
CLIP_C = 16.11809565095832
_INTERPRET = False


def _split_hi_lo(y):
    hi = y.astype(jnp.bfloat16)
    lo = (y - hi.astype(jnp.float32)).astype(jnp.bfloat16)
    return hi, lo


def _mm_bias_body(x_ref, w_ref, b_ref, o_ref):
    o_ref[...] = (
        jnp.dot(x_ref[...], w_ref[...], preferred_element_type=jnp.float32)
        + b_ref[...]
    )


def _mm_bias(x, w, b):
    m, _ = x.shape
    _, n = w.shape
    return pl.pallas_call(
        _mm_bias_body,
        out_shape=jax.ShapeDtypeStruct((m, n), jnp.float32),
        interpret=_INTERPRET,
    )(x, w, b.reshape(1, n))


def _mm_split_body(x_ref, w_ref, hi_ref, lo_ref):
    y = jnp.dot(x_ref[...], w_ref[...], preferred_element_type=jnp.float32)
    hi, lo = _split_hi_lo(y)
    hi_ref[...] = hi
    lo_ref[...] = lo


def _mm_split(x, w):
    m = x.shape[0]
    n = w.shape[1]
    return pl.pallas_call(
        _mm_split_body,
        out_shape=(
            jax.ShapeDtypeStruct((m, n), jnp.bfloat16),
            jax.ShapeDtypeStruct((m, n), jnp.bfloat16),
        ),
        interpret=_INTERPRET,
    )(x, w)


def _split_body(x_ref, hi_ref, lo_ref):
    hi, lo = _split_hi_lo(x_ref[...])
    hi_ref[...] = hi
    lo_ref[...] = lo


def _split(x):
    return pl.pallas_call(
        _split_body,
        out_shape=(
            jax.ShapeDtypeStruct(x.shape, jnp.bfloat16),
            jax.ShapeDtypeStruct(x.shape, jnp.bfloat16),
        ),
        interpret=_INTERPRET,
    )(x)


def _gcn_extract_body(a_ref, yh_ref, yl_ref, b_ref, o_ref, m_ref, v_ref):
    a = a_ref[...]
    mb = (a != 0.0)
    m_ref[...] = mb.astype(jnp.int8)
    v = jnp.max(a, axis=1, keepdims=True)
    v_ref[...] = v
    m16 = mb.astype(jnp.bfloat16)
    s = jnp.dot(m16, yh_ref[...], preferred_element_type=jnp.float32)
    s = s + jnp.dot(m16, yl_ref[...], preferred_element_type=jnp.float32)
    o_ref[...] = jnp.maximum(v * s + b_ref[...], 0.0)


def _gcn_layer1(A, x, W, b):
    yh, yl = _mm_split(x, W)
    return pl.pallas_call(
        _gcn_extract_body,
        grid=(N_BLKS,),
        in_specs=[
            pl.BlockSpec((ROW_BLK, N), lambda i: (i, 0)),
            pl.BlockSpec((N, NHID), lambda i: (0, 0)),
            pl.BlockSpec((N, NHID), lambda i: (0, 0)),
            pl.BlockSpec((1, NHID), lambda i: (0, 0)),
        ],
        out_specs=(
            pl.BlockSpec((ROW_BLK, NHID), lambda i: (i, 0)),
            pl.BlockSpec((ROW_BLK, N), lambda i: (i, 0)),
            pl.BlockSpec((ROW_BLK, 1), lambda i: (i, 0)),
        ),
        out_shape=(
            jax.ShapeDtypeStruct((N, NHID), jnp.float32),
            jax.ShapeDtypeStruct((N, N), jnp.int8),
            jax.ShapeDtypeStruct((N, 1), jnp.float32),
        ),
        interpret=_INTERPRET,
    )(A, yh, yl, b.reshape(1, NHID))


def _gcn_mask_body(relu, residual, m_ref, v_ref, yh_ref, yl_ref, b_ref,
                   x_ref, o_ref):
    m16 = m_ref[...].astype(jnp.bfloat16)
    s = jnp.dot(m16, yh_ref[...], preferred_element_type=jnp.float32)
    s = s + jnp.dot(m16, yl_ref[...], preferred_element_type=jnp.float32)
    s = v_ref[...] * s + b_ref[...]
    if relu:
        s = jnp.maximum(s, 0.0)
    if residual:
        s = s + x_ref[...]
    o_ref[...] = s


def _gcn_layer_masked(mask, v, x, W, b, relu, residual):
    yh, yl = _mm_split(x, W)
    body = functools.partial(_gcn_mask_body, relu, residual)
    return pl.pallas_call(
        body,
        grid=(N_BLKS,),
        in_specs=[
            pl.BlockSpec((ROW_BLK, N), lambda i: (i, 0)),
            pl.BlockSpec((ROW_BLK, 1), lambda i: (i, 0)),
            pl.BlockSpec((N, NHID), lambda i: (0, 0)),
            pl.BlockSpec((N, NHID), lambda i: (0, 0)),
            pl.BlockSpec((1, NHID), lambda i: (0, 0)),
            pl.BlockSpec((ROW_BLK, NHID), lambda i: (i, 0)),
        ],
        out_specs=pl.BlockSpec((ROW_BLK, NHID), lambda i: (i, 0)),
        out_shape=jax.ShapeDtypeStruct((N, NHID), jnp.float32),
        interpret=_INTERPRET,
    )(mask, v, yh, yl, b.reshape(1, NHID), x)


def _bce_body(xh_blk_ref, xl_blk_ref, xh_ref, xl_ref, adj_ref, o_ref):
    i = pl.program_id(0)
    dims = (((1,), (1,)), ((), ()))
    z = jax.lax.dot_general(xh_blk_ref[...], xh_ref[...], dims,
                            preferred_element_type=jnp.float32)
    z = z + jax.lax.dot_general(xh_blk_ref[...], xl_ref[...], dims,
                                preferred_element_type=jnp.float32)
    z = z + jax.lax.dot_general(xl_blk_ref[...], xh_ref[...], dims,
                                preferred_element_type=jnp.float32)
    a = adj_ref[...]
    sp = jnp.minimum(jnp.maximum(z, 0.0) + jnp.log1p(jnp.exp(-jnp.abs(z))),
                     CLIP_C)
    zc = jnp.clip(z, -CLIP_C, CLIP_C)
    part = jnp.sum(sp - a * zc).reshape(1, 1)

    @pl.when(i == 0)
    def _():
        o_ref[...] = jnp.zeros((1, 1), jnp.float32)

    o_ref[...] += part


def _bce_loss(x, adj_full):
    xh, xl = _split(x)
    s = pl.pallas_call(
        _bce_body,
        grid=(N_BLKS,),
        in_specs=[
            pl.BlockSpec((ROW_BLK, NHID), lambda i: (i, 0)),
            pl.BlockSpec((ROW_BLK, NHID), lambda i: (i, 0)),
            pl.BlockSpec((N, NHID), lambda i: (0, 0)),
            pl.BlockSpec((N, NHID), lambda i: (0, 0)),
            pl.BlockSpec((ROW_BLK, N), lambda i: (i, 0)),
        ],
        out_specs=pl.BlockSpec((1, 1), lambda i: (0, 0)),
        out_shape=jax.ShapeDtypeStruct((1, 1), jnp.float32),
        interpret=_INTERPRET,
    )(xh, xl, xh, xl, adj_full)
    return s[0, 0] / (float(N) * float(N))


def _recon_loss_body(r_ref, f_ref, o_ref):
    r = r_ref[...]
    f = f_ref[...]
    rn = jnp.maximum(jnp.sqrt(jnp.sum(r * r, axis=-1, keepdims=True)), 1e-12)
    fn = jnp.maximum(jnp.sqrt(jnp.sum(f * f, axis=-1, keepdims=True)), 1e-12)
    cs = jnp.sum((r / rn) * (f / fn), axis=-1)
    o_ref[...] = jnp.sum((1.0 - cs) ** GAMMA).reshape(1, 1)


def _recon_loss(recon0, X0):
    s = pl.pallas_call(
        _recon_loss_body,
        out_shape=jax.ShapeDtypeStruct((1, 1), jnp.float32),
        interpret=_INTERPRET,
    )(recon0, X0)
    return s[0, 0] / float(X0.shape[0])


def kernel(A, adj_full, X0, X1, X2, fcW0, fcb0, fcW1, fcb1, fcW2, fcb2,
           encW0, encb0, encW1, encb1, decW0, decb0, decW1, decb1,
           fc2W0, fc2b0, fc2W1, fc2b1, fc2W2, fc2b2):
    trans = [
        _mm_bias(X0, fcW0, fcb0),
        _mm_bias(X1, fcW1, fcb1),
        _mm_bias(X2, fcW2, fcb2),
    ]
    x = jnp.concatenate(trans, axis=0)

    x, mask, v = _gcn_layer1(A, x, encW0, encb0)
    x = _gcn_layer_masked(mask, v, x, encW1, encb1, relu=False, residual=True)
    x = _gcn_layer_masked(mask, v, x, decW0, decb0, relu=True, residual=False)
    x = _gcn_layer_masked(mask, v, x, decW1, decb1, relu=False, residual=True)

    n0, n1 = X0.shape[0], X1.shape[0]
    recon0 = _mm_bias(x[:n0], fc2W0, fc2b0)
    recon1 = _mm_bias(x[n0:n0 + n1], fc2W1, fc2b1)
    recon2 = _mm_bias(x[n0 + n1:], fc2W2, fc2b2)

    emb_loss = _bce_loss(x, adj_full)
    recon_loss = _recon_loss(recon0, X0)
    return (recon0, recon1, recon2, emb_loss, recon_loss)


# speedup vs baseline: 1.2060x; 1.2060x over previous
import functools

import jax
import jax.numpy as jnp
from jax.experimental import pallas as pl

N = 11616
NHID = 128
ROW_BLK = 352
N_BLKS = N // ROW_BLK
GAMMA = 2.0
CLIP_C = 16.11809565095832
_INTERPRET = False


def _mm_bias_body(x_ref, w_ref, b_ref, o_ref):
    o_ref[...] = (
        jnp.dot(x_ref[...], w_ref[...], preferred_element_type=jnp.float32)
        + b_ref[...]
    )


def _mm_bias(x, w, b):
    m, _ = x.shape
    _, n = w.shape
    return pl.pallas_call(
        _mm_bias_body,
        out_shape=jax.ShapeDtypeStruct((m, n), jnp.float32),
        interpret=_INTERPRET,
    )(x, w, b.reshape(1, n))


def _mm_bf16_body(x_ref, w_ref, o_ref):
    y = jnp.dot(x_ref[...], w_ref[...], preferred_element_type=jnp.float32)
    o_ref[...] = y.astype(jnp.bfloat16)


def _mm_bf16(x, w):
    m = x.shape[0]
    n = w.shape[1]
    return pl.pallas_call(
        _mm_bf16_body,
        out_shape=jax.ShapeDtypeStruct((m, n), jnp.bfloat16),
        interpret=_INTERPRET,
    )(x, w)


def _gcn_extract_body(a_ref, y_ref, b_ref, o_ref, m_ref, v_ref):
    a = a_ref[...]
    m16 = (a != 0.0).astype(jnp.bfloat16)
    m_ref[...] = m16
    v = jnp.max(a, axis=1, keepdims=True)
    v_ref[...] = v
    s = jnp.dot(m16, y_ref[...], preferred_element_type=jnp.float32)
    o_ref[...] = jnp.maximum(v * s + b_ref[...], 0.0)


def _gcn_layer1(A, x, W, b):
    y = _mm_bf16(x, W)
    return pl.pallas_call(
        _gcn_extract_body,
        grid=(N_BLKS,),
        in_specs=[
            pl.BlockSpec((ROW_BLK, N), lambda i: (i, 0)),
            pl.BlockSpec((N, NHID), lambda i: (0, 0)),
            pl.BlockSpec((1, NHID), lambda i: (0, 0)),
        ],
        out_specs=(
            pl.BlockSpec((ROW_BLK, NHID), lambda i: (i, 0)),
            pl.BlockSpec((ROW_BLK, N), lambda i: (i, 0)),
            pl.BlockSpec((ROW_BLK, 1), lambda i: (i, 0)),
        ),
        out_shape=(
            jax.ShapeDtypeStruct((N, NHID), jnp.float32),
            jax.ShapeDtypeStruct((N, N), jnp.bfloat16),
            jax.ShapeDtypeStruct((N, 1), jnp.float32),
        ),
        interpret=_INTERPRET,
    )(A, y, b.reshape(1, NHID))


def _gcn_mask_body(relu, residual, m_ref, v_ref, y_ref, b_ref, x_ref, o_ref):
    s = jnp.dot(m_ref[...], y_ref[...], preferred_element_type=jnp.float32)
    s = v_ref[...] * s + b_ref[...]
    if relu:
        s = jnp.maximum(s, 0.0)
    if residual:
        s = s + x_ref[...]
    o_ref[...] = s


def _gcn_layer_masked(mask, v, x, W, b, relu, residual):
    y = _mm_bf16(x, W)
    body = functools.partial(_gcn_mask_body, relu, residual)
    return pl.pallas_call(
        body,
        grid=(N_BLKS,),
        in_specs=[
            pl.BlockSpec((ROW_BLK, N), lambda i: (i, 0)),
            pl.BlockSpec((ROW_BLK, 1), lambda i: (i, 0)),
            pl.BlockSpec((N, NHID), lambda i: (0, 0)),
            pl.BlockSpec((1, NHID), lambda i: (0, 0)),
            pl.BlockSpec((ROW_BLK, NHID), lambda i: (i, 0)),
        ],
        out_specs=pl.BlockSpec((ROW_BLK, NHID), lambda i: (i, 0)),
        out_shape=jax.ShapeDtypeStruct((N, NHID), jnp.float32),
        interpret=_INTERPRET,
    )(mask, v, y, b.reshape(1, NHID), x)


def _bce_body(x_blk_ref, x_all_ref, adj_ref, o_ref):
    i = pl.program_id(0)
    z = jax.lax.dot_general(
        x_blk_ref[...], x_all_ref[...],
        (((1,), (1,)), ((), ())),
        preferred_element_type=jnp.float32,
    )
    a = adj_ref[...]
    w = z - (a + a) * z
    sp = jnp.maximum(w, 0.0) + jnp.log1p(jnp.exp(-jnp.abs(w)))
    part = jnp.sum(jnp.minimum(sp, CLIP_C)).reshape(1, 1)

    @pl.when(i == 0)
    def _():
        o_ref[...] = jnp.zeros((1, 1), jnp.float32)

    o_ref[...] += part


def _bce_loss(x, adj_full):
    s = pl.pallas_call(
        _bce_body,
        grid=(N_BLKS,),
        in_specs=[
            pl.BlockSpec((ROW_BLK, NHID), lambda i: (i, 0)),
            pl.BlockSpec((N, NHID), lambda i: (0, 0)),
            pl.BlockSpec((ROW_BLK, N), lambda i: (i, 0)),
        ],
        out_specs=pl.BlockSpec((1, 1), lambda i: (0, 0)),
        out_shape=jax.ShapeDtypeStruct((1, 1), jnp.float32),
        interpret=_INTERPRET,
    )(x, x, adj_full)
    return s[0, 0] / (float(N) * float(N))


def _recon_loss_body(r_ref, f_ref, o_ref):
    r = r_ref[...]
    f = f_ref[...]
    rn = jnp.maximum(jnp.sqrt(jnp.sum(r * r, axis=-1, keepdims=True)), 1e-12)
    fn = jnp.maximum(jnp.sqrt(jnp.sum(f * f, axis=-1, keepdims=True)), 1e-12)
    cs = jnp.sum((r / rn) * (f / fn), axis=-1)
    o_ref[...] = jnp.sum((1.0 - cs) ** GAMMA).reshape(1, 1)


def _recon_loss(recon0, X0):
    s = pl.pallas_call(
        _recon_loss_body,
        out_shape=jax.ShapeDtypeStruct((1, 1), jnp.float32),
        interpret=_INTERPRET,
    )(recon0, X0)
    return s[0, 0] / float(X0.shape[0])


def kernel(A, adj_full, X0, X1, X2, fcW0, fcb0, fcW1, fcb1, fcW2, fcb2,
           encW0, encb0, encW1, encb1, decW0, decb0, decW1, decb1,
           fc2W0, fc2b0, fc2W1, fc2b1, fc2W2, fc2b2):
    trans = [
        _mm_bias(X0, fcW0, fcb0),
        _mm_bias(X1, fcW1, fcb1),
        _mm_bias(X2, fcW2, fcb2),
    ]
    x = jnp.concatenate(trans, axis=0)

    x, mask, v = _gcn_layer1(A, x, encW0, encb0)
    x = _gcn_layer_masked(mask, v, x, encW1, encb1, relu=False, residual=True)
    x = _gcn_layer_masked(mask, v, x, decW0, decb0, relu=True, residual=False)
    x = _gcn_layer_masked(mask, v, x, decW1, decb1, relu=False, residual=True)

    n0, n1 = X0.shape[0], X1.shape[0]
    recon0 = _mm_bias(x[:n0], fc2W0, fc2b0)
    recon1 = _mm_bias(x[n0:n0 + n1], fc2W1, fc2b1)
    recon2 = _mm_bias(x[n0 + n1:], fc2W2, fc2b2)

    emb_loss = _bce_loss(x, adj_full)
    recon_loss = _recon_loss(recon0, X0)
    return (recon0, recon1, recon2, emb_loss, recon_loss)


# speedup vs baseline: 1.3666x; 1.1332x over previous
import functools

import jax
import jax.numpy as jnp
from jax.experimental import pallas as pl

N = 11616
NHID = 128
ROW_BLK = 352
N_BLKS = N // ROW_BLK
GAMMA = 2.0
CLIP_C = 16.11809565095832
_INTERPRET = False


def _mm_bias_body(x_ref, w_ref, b_ref, o_ref):
    o_ref[...] = (
        jnp.dot(x_ref[...], w_ref[...], preferred_element_type=jnp.float32)
        + b_ref[...]
    )


def _mm_bias(x, w, b):
    m, _ = x.shape
    _, n = w.shape
    return pl.pallas_call(
        _mm_bias_body,
        out_shape=jax.ShapeDtypeStruct((m, n), jnp.float32),
        interpret=_INTERPRET,
    )(x, w, b.reshape(1, n))


def _mm_bf16_body(x_ref, w_ref, o_ref):
    y = jnp.dot(x_ref[...], w_ref[...], preferred_element_type=jnp.float32)
    o_ref[...] = y.astype(jnp.bfloat16)


def _mm_bf16(x, w):
    m = x.shape[0]
    n = w.shape[1]
    return pl.pallas_call(
        _mm_bf16_body,
        out_shape=jax.ShapeDtypeStruct((m, n), jnp.bfloat16),
        interpret=_INTERPRET,
    )(x, w)


def _gcn_extract_body(a_ref, y_ref, b_ref, o_ref, m_ref, v_ref):
    a = a_ref[...]
    mb = a != 0.0
    m_ref[...] = mb.astype(jnp.int8)
    v = jnp.max(a, axis=1, keepdims=True)
    v_ref[...] = v
    s = jnp.dot(mb.astype(jnp.bfloat16), y_ref[...],
                preferred_element_type=jnp.float32)
    o_ref[...] = jnp.maximum(v * s + b_ref[...], 0.0)


def _gcn_layer1(A, x, W, b):
    y = _mm_bf16(x, W)
    return pl.pallas_call(
        _gcn_extract_body,
        grid=(N_BLKS,),
        in_specs=[
            pl.BlockSpec((ROW_BLK, N), lambda i: (i, 0)),
            pl.BlockSpec((N, NHID), lambda i: (0, 0)),
            pl.BlockSpec((1, NHID), lambda i: (0, 0)),
        ],
        out_specs=(
            pl.BlockSpec((ROW_BLK, NHID), lambda i: (i, 0)),
            pl.BlockSpec((ROW_BLK, N), lambda i: (i, 0)),
            pl.BlockSpec((ROW_BLK, 1), lambda i: (i, 0)),
        ),
        out_shape=(
            jax.ShapeDtypeStruct((N, NHID), jnp.float32),
            jax.ShapeDtypeStruct((N, N), jnp.int8),
            jax.ShapeDtypeStruct((N, 1), jnp.float32),
        ),
        interpret=_INTERPRET,
    )(A, y, b.reshape(1, NHID))


def _gcn_mask_body(relu, residual, m_ref, v_ref, y_ref, b_ref, x_ref, o_ref):
    s = jnp.dot(m_ref[...].astype(jnp.bfloat16), y_ref[...],
                preferred_element_type=jnp.float32)
    s = v_ref[...] * s + b_ref[...]
    if relu:
        s = jnp.maximum(s, 0.0)
    if residual:
        s = s + x_ref[...]
    o_ref[...] = s


def _gcn_layer_masked(mask, v, x, W, b, relu, residual):
    y = _mm_bf16(x, W)
    body = functools.partial(_gcn_mask_body, relu, residual)
    return pl.pallas_call(
        body,
        grid=(N_BLKS,),
        in_specs=[
            pl.BlockSpec((ROW_BLK, N), lambda i: (i, 0)),
            pl.BlockSpec((ROW_BLK, 1), lambda i: (i, 0)),
            pl.BlockSpec((N, NHID), lambda i: (0, 0)),
            pl.BlockSpec((1, NHID), lambda i: (0, 0)),
            pl.BlockSpec((ROW_BLK, NHID), lambda i: (i, 0)),
        ],
        out_specs=pl.BlockSpec((ROW_BLK, NHID), lambda i: (i, 0)),
        out_shape=jax.ShapeDtypeStruct((N, NHID), jnp.float32),
        interpret=_INTERPRET,
    )(mask, v, y, b.reshape(1, NHID), x)


def _bce_body(x_blk_ref, x_all_ref, adj_ref, o_ref):
    i = pl.program_id(0)
    z = jax.lax.dot_general(
        x_blk_ref[...], x_all_ref[...],
        (((1,), (1,)), ((), ())),
        preferred_element_type=jnp.float32,
    )
    a = adj_ref[...]
    w = z - (a + a) * z
    sp = jnp.maximum(w, 0.0) + jnp.log1p(jnp.exp(-jnp.abs(w)))
    part = jnp.sum(jnp.minimum(sp, CLIP_C)).reshape(1, 1)

    @pl.when(i == 0)
    def _():
        o_ref[...] = jnp.zeros((1, 1), jnp.float32)

    o_ref[...] += part


def _bce_loss(x, adj_full):
    s = pl.pallas_call(
        _bce_body,
        grid=(N_BLKS,),
        in_specs=[
            pl.BlockSpec((ROW_BLK, NHID), lambda i: (i, 0)),
            pl.BlockSpec((N, NHID), lambda i: (0, 0)),
            pl.BlockSpec((ROW_BLK, N), lambda i: (i, 0)),
        ],
        out_specs=pl.BlockSpec((1, 1), lambda i: (0, 0)),
        out_shape=jax.ShapeDtypeStruct((1, 1), jnp.float32),
        interpret=_INTERPRET,
    )(x, x, adj_full)
    return s[0, 0] / (float(N) * float(N))


def _recon_loss_body(r_ref, f_ref, o_ref):
    r = r_ref[...]
    f = f_ref[...]
    rn = jnp.maximum(jnp.sqrt(jnp.sum(r * r, axis=-1, keepdims=True)), 1e-12)
    fn = jnp.maximum(jnp.sqrt(jnp.sum(f * f, axis=-1, keepdims=True)), 1e-12)
    cs = jnp.sum((r / rn) * (f / fn), axis=-1)
    o_ref[...] = jnp.sum((1.0 - cs) ** GAMMA).reshape(1, 1)


def _recon_loss(recon0, X0):
    s = pl.pallas_call(
        _recon_loss_body,
        out_shape=jax.ShapeDtypeStruct((1, 1), jnp.float32),
        interpret=_INTERPRET,
    )(recon0, X0)
    return s[0, 0] / float(X0.shape[0])


def kernel(A, adj_full, X0, X1, X2, fcW0, fcb0, fcW1, fcb1, fcW2, fcb2,
           encW0, encb0, encW1, encb1, decW0, decb0, decW1, decb1,
           fc2W0, fc2b0, fc2W1, fc2b1, fc2W2, fc2b2):
    trans = [
        _mm_bias(X0, fcW0, fcb0),
        _mm_bias(X1, fcW1, fcb1),
        _mm_bias(X2, fcW2, fcb2),
    ]
    x = jnp.concatenate(trans, axis=0)

    x, mask, v = _gcn_layer1(A, x, encW0, encb0)
    x = _gcn_layer_masked(mask, v, x, encW1, encb1, relu=False, residual=True)
    x = _gcn_layer_masked(mask, v, x, decW0, decb0, relu=True, residual=False)
    x = _gcn_layer_masked(mask, v, x, decW1, decb1, relu=False, residual=True)

    n0, n1 = X0.shape[0], X1.shape[0]
    recon0 = _mm_bias(x[:n0], fc2W0, fc2b0)
    recon1 = _mm_bias(x[n0:n0 + n1], fc2W1, fc2b1)
    recon2 = _mm_bias(x[n0 + n1:], fc2W2, fc2b2)

    emb_loss = _bce_loss(x, adj_full)
    recon_loss = _recon_loss(recon0, X0)
    return (recon0, recon1, recon2, emb_loss, recon_loss)


# speedup vs baseline: 1.4815x; 1.0841x over previous
import functools

import jax
import jax.numpy as jnp
from jax.experimental import pallas as pl

N = 11616
NHID = 128
ROW_BLK = 352
N_BLKS = N // ROW_BLK
GAMMA = 2.0
CLIP_C = 16.11809565095832
_INTERPRET = False


def _mm_bias_body(x_ref, w_ref, b_ref, o_ref):
    o_ref[...] = (
        jnp.dot(x_ref[...], w_ref[...], preferred_element_type=jnp.float32)
        + b_ref[...]
    )


def _mm_bias(x, w, b):
    m, _ = x.shape
    _, n = w.shape
    return pl.pallas_call(
        _mm_bias_body,
        out_shape=jax.ShapeDtypeStruct((m, n), jnp.float32),
        interpret=_INTERPRET,
    )(x, w, b.reshape(1, n))


def _mm_bf16_body(x_ref, w_ref, o_ref):
    y = jnp.dot(x_ref[...], w_ref[...], preferred_element_type=jnp.float32)
    o_ref[...] = y.astype(jnp.bfloat16)


def _mm_bf16(x, w):
    m = x.shape[0]
    n = w.shape[1]
    return pl.pallas_call(
        _mm_bf16_body,
        out_shape=jax.ShapeDtypeStruct((m, n), jnp.bfloat16),
        interpret=_INTERPRET,
    )(x, w)


def _gcn_extract_body(a_ref, y_ref, b_ref, o_ref, m_ref, v_ref):
    a = a_ref[...]
    mb = a != 0.0
    m_ref[...] = mb.astype(jnp.int8)
    v = jnp.max(a, axis=1, keepdims=True)
    v_ref[...] = v
    s = jnp.dot(mb.astype(jnp.bfloat16), y_ref[...],
                preferred_element_type=jnp.float32)
    o_ref[...] = jnp.maximum(v * s + b_ref[...], 0.0)


def _gcn_layer1(A, x, W, b):
    y = _mm_bf16(x, W)
    return pl.pallas_call(
        _gcn_extract_body,
        grid=(N_BLKS,),
        in_specs=[
            pl.BlockSpec((ROW_BLK, N), lambda i: (i, 0)),
            pl.BlockSpec((N, NHID), lambda i: (0, 0)),
            pl.BlockSpec((1, NHID), lambda i: (0, 0)),
        ],
        out_specs=(
            pl.BlockSpec((ROW_BLK, NHID), lambda i: (i, 0)),
            pl.BlockSpec((ROW_BLK, N), lambda i: (i, 0)),
            pl.BlockSpec((ROW_BLK, 1), lambda i: (i, 0)),
        ),
        out_shape=(
            jax.ShapeDtypeStruct((N, NHID), jnp.float32),
            jax.ShapeDtypeStruct((N, N), jnp.int8),
            jax.ShapeDtypeStruct((N, 1), jnp.float32),
        ),
        interpret=_INTERPRET,
    )(A, y, b.reshape(1, NHID))


def _gcn_mask_body(relu, residual, m_ref, v_ref, y_ref, b_ref, x_ref, o_ref):
    s = jnp.dot(m_ref[...].astype(jnp.bfloat16), y_ref[...],
                preferred_element_type=jnp.float32)
    s = v_ref[...] * s + b_ref[...]
    if relu:
        s = jnp.maximum(s, 0.0)
    if residual:
        s = s + x_ref[...]
    o_ref[...] = s


def _gcn_layer_masked(mask, v, x, W, b, relu, residual):
    y = _mm_bf16(x, W)
    body = functools.partial(_gcn_mask_body, relu, residual)
    return pl.pallas_call(
        body,
        grid=(N_BLKS,),
        in_specs=[
            pl.BlockSpec((ROW_BLK, N), lambda i: (i, 0)),
            pl.BlockSpec((ROW_BLK, 1), lambda i: (i, 0)),
            pl.BlockSpec((N, NHID), lambda i: (0, 0)),
            pl.BlockSpec((1, NHID), lambda i: (0, 0)),
            pl.BlockSpec((ROW_BLK, NHID), lambda i: (i, 0)),
        ],
        out_specs=pl.BlockSpec((ROW_BLK, NHID), lambda i: (i, 0)),
        out_shape=jax.ShapeDtypeStruct((N, NHID), jnp.float32),
        interpret=_INTERPRET,
    )(mask, v, y, b.reshape(1, NHID), x)


def _bce_body(x_blk_ref, x_all_ref, adj_ref, o_ref):
    i = pl.program_id(0)
    z = jax.lax.dot_general(
        x_blk_ref[...], x_all_ref[...],
        (((1,), (1,)), ((), ())),
        preferred_element_type=jnp.float32,
    )
    a = adj_ref[...]
    w = (z - (a + a) * z).astype(jnp.bfloat16)
    sp = jnp.maximum(w, 0.0) + jnp.log1p(jnp.exp(-jnp.abs(w)))
    sp32 = jnp.minimum(sp.astype(jnp.float32), CLIP_C)
    part = jnp.sum(sp32).reshape(1, 1)

    @pl.when(i == 0)
    def _():
        o_ref[...] = jnp.zeros((1, 1), jnp.float32)

    o_ref[...] += part


def _bce_loss(x, adj_full):
    s = pl.pallas_call(
        _bce_body,
        grid=(N_BLKS,),
        in_specs=[
            pl.BlockSpec((ROW_BLK, NHID), lambda i: (i, 0)),
            pl.BlockSpec((N, NHID), lambda i: (0, 0)),
            pl.BlockSpec((ROW_BLK, N), lambda i: (i, 0)),
        ],
        out_specs=pl.BlockSpec((1, 1), lambda i: (0, 0)),
        out_shape=jax.ShapeDtypeStruct((1, 1), jnp.float32),
        interpret=_INTERPRET,
    )(x, x, adj_full)
    return s[0, 0] / (float(N) * float(N))


def _recon_loss_body(r_ref, f_ref, o_ref):
    r = r_ref[...]
    f = f_ref[...]
    rn = jnp.maximum(jnp.sqrt(jnp.sum(r * r, axis=-1, keepdims=True)), 1e-12)
    fn = jnp.maximum(jnp.sqrt(jnp.sum(f * f, axis=-1, keepdims=True)), 1e-12)
    cs = jnp.sum((r / rn) * (f / fn), axis=-1)
    o_ref[...] = jnp.sum((1.0 - cs) ** GAMMA).reshape(1, 1)


def _recon_loss(recon0, X0):
    s = pl.pallas_call(
        _recon_loss_body,
        out_shape=jax.ShapeDtypeStruct((1, 1), jnp.float32),
        interpret=_INTERPRET,
    )(recon0, X0)
    return s[0, 0] / float(X0.shape[0])


def kernel(A, adj_full, X0, X1, X2, fcW0, fcb0, fcW1, fcb1, fcW2, fcb2,
           encW0, encb0, encW1, encb1, decW0, decb0, decW1, decb1,
           fc2W0, fc2b0, fc2W1, fc2b1, fc2W2, fc2b2):
    trans = [
        _mm_bias(X0, fcW0, fcb0),
        _mm_bias(X1, fcW1, fcb1),
        _mm_bias(X2, fcW2, fcb2),
    ]
    x = jnp.concatenate(trans, axis=0)

    x, mask, v = _gcn_layer1(A, x, encW0, encb0)
    x = _gcn_layer_masked(mask, v, x, encW1, encb1, relu=False, residual=True)
    x = _gcn_layer_masked(mask, v, x, decW0, decb0, relu=True, residual=False)
    x = _gcn_layer_masked(mask, v, x, decW1, decb1, relu=False, residual=True)

    n0, n1 = X0.shape[0], X1.shape[0]
    recon0 = _mm_bias(x[:n0], fc2W0, fc2b0)
    recon1 = _mm_bias(x[n0:n0 + n1], fc2W1, fc2b1)
    recon2 = _mm_bias(x[n0 + n1:], fc2W2, fc2b2)

    emb_loss = _bce_loss(x, adj_full)
    recon_loss = _recon_loss(recon0, X0)
    return (recon0, recon1, recon2, emb_loss, recon_loss)


# speedup vs baseline: 1.5143x; 1.0221x over previous
import functools

import jax
import jax.numpy as jnp
from jax.experimental import pallas as pl

N = 11616
NHID = 128
ROW_BLK = 352
N_BLKS = N // ROW_BLK
GAMMA = 2.0
CLIP_C = 16.11809565095832
_INTERPRET = False


def _encode_body(x_ref, w_ref, b_ref, w2_ref, y_ref):
    t = (jnp.dot(x_ref[...], w_ref[...], preferred_element_type=jnp.float32)
         + b_ref[...])
    y = jnp.dot(t, w2_ref[...], preferred_element_type=jnp.float32)
    y_ref[...] = y.astype(jnp.bfloat16)


def _encode(x, w, b, w2):
    m = x.shape[0]
    return pl.pallas_call(
        _encode_body,
        out_shape=jax.ShapeDtypeStruct((m, NHID), jnp.bfloat16),
        interpret=_INTERPRET,
    )(x, w, b.reshape(1, NHID), w2)


def _mm_bias_body(x_ref, w_ref, b_ref, o_ref):
    o_ref[...] = (
        jnp.dot(x_ref[...], w_ref[...], preferred_element_type=jnp.float32)
        + b_ref[...]
    )


def _mm_bias(x, w, b):
    m, _ = x.shape
    _, n = w.shape
    return pl.pallas_call(
        _mm_bias_body,
        out_shape=jax.ShapeDtypeStruct((m, n), jnp.float32),
        interpret=_INTERPRET,
    )(x, w, b.reshape(1, n))


def _gcn_extract_body(a_ref, y_ref, b_ref, w2_ref, o_ref, m_ref, v_ref,
                      y2_ref):
    a = a_ref[...]
    mb = a != 0.0
    m_ref[...] = mb.astype(jnp.int8)
    v = jnp.max(a, axis=1, keepdims=True)
    v_ref[...] = v
    s = jnp.dot(mb.astype(jnp.bfloat16), y_ref[...],
                preferred_element_type=jnp.float32)
    o = jnp.maximum(v * s + b_ref[...], 0.0)
    o_ref[...] = o
    y2 = jnp.dot(o, w2_ref[...], preferred_element_type=jnp.float32)
    y2_ref[...] = y2.astype(jnp.bfloat16)


def _gcn_layer1(A, y, b, w2):
    return pl.pallas_call(
        _gcn_extract_body,
        grid=(N_BLKS,),
        in_specs=[
            pl.BlockSpec((ROW_BLK, N), lambda i: (i, 0)),
            pl.BlockSpec((N, NHID), lambda i: (0, 0)),
            pl.BlockSpec((1, NHID), lambda i: (0, 0)),
            pl.BlockSpec((NHID, NHID), lambda i: (0, 0)),
        ],
        out_specs=(
            pl.BlockSpec((ROW_BLK, NHID), lambda i: (i, 0)),
            pl.BlockSpec((ROW_BLK, N), lambda i: (i, 0)),
            pl.BlockSpec((ROW_BLK, 1), lambda i: (i, 0)),
            pl.BlockSpec((ROW_BLK, NHID), lambda i: (i, 0)),
        ),
        out_shape=(
            jax.ShapeDtypeStruct((N, NHID), jnp.float32),
            jax.ShapeDtypeStruct((N, N), jnp.int8),
            jax.ShapeDtypeStruct((N, 1), jnp.float32),
            jax.ShapeDtypeStruct((N, NHID), jnp.bfloat16),
        ),
        interpret=_INTERPRET,
    )(A, y, b.reshape(1, NHID), w2)


def _gcn_mask_body(relu, residual, last, m_ref, v_ref, y_ref, b_ref,
                   x_ref, w2_ref, o_ref, y2_ref):
    s = jnp.dot(m_ref[...].astype(jnp.bfloat16), y_ref[...],
                preferred_element_type=jnp.float32)
    s = v_ref[...] * s + b_ref[...]
    if relu:
        s = jnp.maximum(s, 0.0)
    if residual:
        s = s + x_ref[...]
    o_ref[...] = s
    if not last:
        y2 = jnp.dot(s, w2_ref[...], preferred_element_type=jnp.float32)
        y2_ref[...] = y2.astype(jnp.bfloat16)


def _gcn_layer_masked(mask, v, y, b, x, w2, relu, residual):
    last = w2 is None
    body = functools.partial(_gcn_mask_body, relu, residual, last)
    if last:
        w2 = jnp.zeros((NHID, NHID), jnp.float32)
    out = pl.pallas_call(
        body,
        grid=(N_BLKS,),
        in_specs=[
            pl.BlockSpec((ROW_BLK, N), lambda i: (i, 0)),
            pl.BlockSpec((ROW_BLK, 1), lambda i: (i, 0)),
            pl.BlockSpec((N, NHID), lambda i: (0, 0)),
            pl.BlockSpec((1, NHID), lambda i: (0, 0)),
            pl.BlockSpec((ROW_BLK, NHID), lambda i: (i, 0)),
            pl.BlockSpec((NHID, NHID), lambda i: (0, 0)),
        ],
        out_specs=(
            pl.BlockSpec((ROW_BLK, NHID), lambda i: (i, 0)),
            pl.BlockSpec((ROW_BLK, NHID), lambda i: (i, 0)),
        ),
        out_shape=(
            jax.ShapeDtypeStruct((N, NHID), jnp.float32),
            jax.ShapeDtypeStruct((N, NHID), jnp.bfloat16),
        ),
        interpret=_INTERPRET,
    )(mask, v, y, b.reshape(1, NHID), x, w2)
    return out


def _bce_body(x_blk_ref, x_all_ref, adj_ref, o_ref):
    i = pl.program_id(0)
    z = jax.lax.dot_general(
        x_blk_ref[...], x_all_ref[...],
        (((1,), (1,)), ((), ())),
        preferred_element_type=jnp.float32,
    )
    a = adj_ref[...]
    w = (z - (a + a) * z).astype(jnp.bfloat16)
    sp = jnp.maximum(w, 0.0) + jnp.log1p(jnp.exp(-jnp.abs(w)))
    sp32 = jnp.minimum(sp.astype(jnp.float32), CLIP_C)
    part = jnp.sum(sp32).reshape(1, 1)

    @pl.when(i == 0)
    def _():
        o_ref[...] = jnp.zeros((1, 1), jnp.float32)

    o_ref[...] += part


def _bce_loss(x, adj_full):
    s = pl.pallas_call(
        _bce_body,
        grid=(N_BLKS,),
        in_specs=[
            pl.BlockSpec((ROW_BLK, NHID), lambda i: (i, 0)),
            pl.BlockSpec((N, NHID), lambda i: (0, 0)),
            pl.BlockSpec((ROW_BLK, N), lambda i: (i, 0)),
        ],
        out_specs=pl.BlockSpec((1, 1), lambda i: (0, 0)),
        out_shape=jax.ShapeDtypeStruct((1, 1), jnp.float32),
        interpret=_INTERPRET,
    )(x, x, adj_full)
    return s[0, 0] / (float(N) * float(N))


def _head0_body(x_ref, w_ref, b_ref, f_ref, o_ref, l_ref):
    r = (jnp.dot(x_ref[...], w_ref[...], preferred_element_type=jnp.float32)
         + b_ref[...])
    o_ref[...] = r
    f = f_ref[...]
    rn = jnp.maximum(jnp.sqrt(jnp.sum(r * r, axis=-1, keepdims=True)), 1e-12)
    fn = jnp.maximum(jnp.sqrt(jnp.sum(f * f, axis=-1, keepdims=True)), 1e-12)
    cs = jnp.sum((r / rn) * (f / fn), axis=-1)
    l_ref[...] = jnp.mean((1.0 - cs) ** GAMMA).reshape(1, 1)


def _head0(x0emb, w, b, X0):
    m, n = X0.shape
    return pl.pallas_call(
        _head0_body,
        out_shape=(
            jax.ShapeDtypeStruct((m, n), jnp.float32),
            jax.ShapeDtypeStruct((1, 1), jnp.float32),
        ),
        interpret=_INTERPRET,
    )(x0emb, w, b.reshape(1, n), X0)


def kernel(A, adj_full, X0, X1, X2, fcW0, fcb0, fcW1, fcb1, fcW2, fcb2,
           encW0, encb0, encW1, encb1, decW0, decb0, decW1, decb1,
           fc2W0, fc2b0, fc2W1, fc2b1, fc2W2, fc2b2):
    y0 = jnp.concatenate([
        _encode(X0, fcW0, fcb0, encW0),
        _encode(X1, fcW1, fcb1, encW0),
        _encode(X2, fcW2, fcb2, encW0),
    ], axis=0)

    x, mask, v, y = _gcn_layer1(A, y0, encb0, encW1)
    x, y = _gcn_layer_masked(mask, v, y, encb1, x, decW0,
                             relu=False, residual=True)
    x, y = _gcn_layer_masked(mask, v, y, decb0, x, decW1,
                             relu=True, residual=False)
    x, _ = _gcn_layer_masked(mask, v, y, decb1, x, None,
                             relu=False, residual=True)

    n0, n1 = X0.shape[0], X1.shape[0]
    recon0, recon_loss = _head0(x[:n0], fc2W0, fc2b0, X0)
    recon1 = _mm_bias(x[n0:n0 + n1], fc2W1, fc2b1)
    recon2 = _mm_bias(x[n0 + n1:], fc2W2, fc2b2)

    emb_loss = _bce_loss(x, adj_full)
    return (recon0, recon1, recon2, emb_loss, recon_loss[0, 0])


# speedup vs baseline: 1.5232x; 1.0058x over previous
import functools

import jax
import jax.numpy as jnp
from jax.experimental import pallas as pl

N = 11616
NHID = 128
ROW_BLK = 352
N_BLKS = N // ROW_BLK
GAMMA = 2.0
CLIP_C = 16.11809565095832
_INTERPRET = False


def _encode_body(x0_ref, x1_ref, x2_ref, w0_ref, w1_ref, w2w_ref,
                 b0_ref, b1_ref, b2_ref, we_ref, y0_ref, y1_ref, y2_ref):
    we = we_ref[...]
    for x_ref, w_ref, b_ref, y_ref in (
            (x0_ref, w0_ref, b0_ref, y0_ref),
            (x1_ref, w1_ref, b1_ref, y1_ref),
            (x2_ref, w2w_ref, b2_ref, y2_ref)):
        t = (jnp.dot(x_ref[...], w_ref[...],
                     preferred_element_type=jnp.float32) + b_ref[...])
        y = jnp.dot(t, we, preferred_element_type=jnp.float32)
        y_ref[...] = y.astype(jnp.bfloat16)


def _encode_all(X0, X1, X2, fcW, fcb, encW0):
    return pl.pallas_call(
        _encode_body,
        out_shape=tuple(
            jax.ShapeDtypeStruct((x.shape[0], NHID), jnp.bfloat16)
            for x in (X0, X1, X2)),
        interpret=_INTERPRET,
    )(X0, X1, X2, fcW[0], fcW[1], fcW[2],
      fcb[0].reshape(1, NHID), fcb[1].reshape(1, NHID),
      fcb[2].reshape(1, NHID), encW0)


def _mm_bias_body(x_ref, w_ref, b_ref, o_ref):
    o_ref[...] = (
        jnp.dot(x_ref[...], w_ref[...], preferred_element_type=jnp.float32)
        + b_ref[...]
    )


def _mm_bias(x, w, b):
    m, _ = x.shape
    _, n = w.shape
    return pl.pallas_call(
        _mm_bias_body,
        out_shape=jax.ShapeDtypeStruct((m, n), jnp.float32),
        interpret=_INTERPRET,
    )(x, w, b.reshape(1, n))


def _gcn_extract_body(a_ref, y_ref, b_ref, w2_ref, o_ref, m_ref, v_ref,
                      y2_ref):
    a = a_ref[...]
    mb = a != 0.0
    m_ref[...] = mb.astype(jnp.int8)
    v = jnp.max(a, axis=1, keepdims=True)
    v_ref[...] = v
    s = jnp.dot(mb.astype(jnp.bfloat16), y_ref[...],
                preferred_element_type=jnp.float32)
    o = jnp.maximum(v * s + b_ref[...], 0.0)
    o_ref[...] = o
    y2 = jnp.dot(o, w2_ref[...], preferred_element_type=jnp.float32)
    y2_ref[...] = y2.astype(jnp.bfloat16)


def _gcn_layer1(A, y, b, w2):
    return pl.pallas_call(
        _gcn_extract_body,
        grid=(N_BLKS,),
        in_specs=[
            pl.BlockSpec((ROW_BLK, N), lambda i: (i, 0)),
            pl.BlockSpec((N, NHID), lambda i: (0, 0)),
            pl.BlockSpec((1, NHID), lambda i: (0, 0)),
            pl.BlockSpec((NHID, NHID), lambda i: (0, 0)),
        ],
        out_specs=(
            pl.BlockSpec((ROW_BLK, NHID), lambda i: (i, 0)),
            pl.BlockSpec((ROW_BLK, N), lambda i: (i, 0)),
            pl.BlockSpec((ROW_BLK, 1), lambda i: (i, 0)),
            pl.BlockSpec((ROW_BLK, NHID), lambda i: (i, 0)),
        ),
        out_shape=(
            jax.ShapeDtypeStruct((N, NHID), jnp.float32),
            jax.ShapeDtypeStruct((N, N), jnp.int8),
            jax.ShapeDtypeStruct((N, 1), jnp.float32),
            jax.ShapeDtypeStruct((N, NHID), jnp.bfloat16),
        ),
        interpret=_INTERPRET,
    )(A, y, b.reshape(1, NHID), w2)


def _gcn_mask_body(relu, residual, last, m_ref, v_ref, y_ref, b_ref,
                   x_ref, w2_ref, o_ref, y2_ref):
    s = jnp.dot(m_ref[...].astype(jnp.bfloat16), y_ref[...],
                preferred_element_type=jnp.float32)
    s = v_ref[...] * s + b_ref[...]
    if relu:
        s = jnp.maximum(s, 0.0)
    if residual:
        s = s + x_ref[...]
    o_ref[...] = s
    if not last:
        y2 = jnp.dot(s, w2_ref[...], preferred_element_type=jnp.float32)
        y2_ref[...] = y2.astype(jnp.bfloat16)


def _gcn_layer_masked(mask, v, y, b, x, w2, relu, residual):
    last = w2 is None
    body = functools.partial(_gcn_mask_body, relu, residual, last)
    if last:
        w2 = jnp.zeros((NHID, NHID), jnp.float32)
    out = pl.pallas_call(
        body,
        grid=(N_BLKS,),
        in_specs=[
            pl.BlockSpec((ROW_BLK, N), lambda i: (i, 0)),
            pl.BlockSpec((ROW_BLK, 1), lambda i: (i, 0)),
            pl.BlockSpec((N, NHID), lambda i: (0, 0)),
            pl.BlockSpec((1, NHID), lambda i: (0, 0)),
            pl.BlockSpec((ROW_BLK, NHID), lambda i: (i, 0)),
            pl.BlockSpec((NHID, NHID), lambda i: (0, 0)),
        ],
        out_specs=(
            pl.BlockSpec((ROW_BLK, NHID), lambda i: (i, 0)),
            pl.BlockSpec((ROW_BLK, NHID), lambda i: (i, 0)),
        ),
        out_shape=(
            jax.ShapeDtypeStruct((N, NHID), jnp.float32),
            jax.ShapeDtypeStruct((N, NHID), jnp.bfloat16),
        ),
        interpret=_INTERPRET,
    )(mask, v, y, b.reshape(1, NHID), x, w2)
    return out


def _bce_body(x_blk_ref, x_all_ref, adj_ref, o_ref):
    i = pl.program_id(0)
    z = jax.lax.dot_general(
        x_blk_ref[...], x_all_ref[...],
        (((1,), (1,)), ((), ())),
        preferred_element_type=jnp.float32,
    )
    a = adj_ref[...]
    w = (z - (a + a) * z).astype(jnp.bfloat16)
    sp = jnp.maximum(w, 0.0) + jnp.log1p(jnp.exp(-jnp.abs(w)))
    sp32 = jnp.minimum(sp.astype(jnp.float32), CLIP_C)
    part = jnp.sum(sp32).reshape(1, 1)

    @pl.when(i == 0)
    def _():
        o_ref[...] = jnp.zeros((1, 1), jnp.float32)

    o_ref[...] += part


def _bce_loss(x, adj_full):
    s = pl.pallas_call(
        _bce_body,
        grid=(N_BLKS,),
        in_specs=[
            pl.BlockSpec((ROW_BLK, NHID), lambda i: (i, 0)),
            pl.BlockSpec((N, NHID), lambda i: (0, 0)),
            pl.BlockSpec((ROW_BLK, N), lambda i: (i, 0)),
        ],
        out_specs=pl.BlockSpec((1, 1), lambda i: (0, 0)),
        out_shape=jax.ShapeDtypeStruct((1, 1), jnp.float32),
        interpret=_INTERPRET,
    )(x, x, adj_full)
    return s[0, 0] / (float(N) * float(N))


def _heads_body(e0_ref, e1_ref, e2_ref, w0_ref, w1_ref, w2_ref,
                b0_ref, b1_ref, b2_ref, f_ref,
                o0_ref, o1_ref, o2_ref, l_ref):
    r = (jnp.dot(e0_ref[...], w0_ref[...],
                 preferred_element_type=jnp.float32) + b0_ref[...])
    o0_ref[...] = r
    f = f_ref[...]
    rn = jnp.maximum(jnp.sqrt(jnp.sum(r * r, axis=-1, keepdims=True)), 1e-12)
    fn = jnp.maximum(jnp.sqrt(jnp.sum(f * f, axis=-1, keepdims=True)), 1e-12)
    cs = jnp.sum((r / rn) * (f / fn), axis=-1)
    l_ref[...] = jnp.mean((1.0 - cs) ** GAMMA).reshape(1, 1)
    o1_ref[...] = (jnp.dot(e1_ref[...], w1_ref[...],
                           preferred_element_type=jnp.float32) + b1_ref[...])
    o2_ref[...] = (jnp.dot(e2_ref[...], w2_ref[...],
                           preferred_element_type=jnp.float32) + b2_ref[...])


def _heads(e0, e1, e2, fc2W, fc2b, X0):
    fo = X0.shape[1]
    return pl.pallas_call(
        _heads_body,
        out_shape=(
            jax.ShapeDtypeStruct((e0.shape[0], fc2W[0].shape[1]), jnp.float32),
            jax.ShapeDtypeStruct((e1.shape[0], fc2W[1].shape[1]), jnp.float32),
            jax.ShapeDtypeStruct((e2.shape[0], fc2W[2].shape[1]), jnp.float32),
            jax.ShapeDtypeStruct((1, 1), jnp.float32),
        ),
        interpret=_INTERPRET,
    )(e0, e1, e2, fc2W[0], fc2W[1], fc2W[2],
      fc2b[0].reshape(1, fc2W[0].shape[1]),
      fc2b[1].reshape(1, fc2W[1].shape[1]),
      fc2b[2].reshape(1, fc2W[2].shape[1]), X0)


def kernel(A, adj_full, X0, X1, X2, fcW0, fcb0, fcW1, fcb1, fcW2, fcb2,
           encW0, encb0, encW1, encb1, decW0, decb0, decW1, decb1,
           fc2W0, fc2b0, fc2W1, fc2b1, fc2W2, fc2b2):
    y0 = jnp.concatenate(
        _encode_all(X0, X1, X2, (fcW0, fcW1, fcW2), (fcb0, fcb1, fcb2),
                    encW0),
        axis=0)

    x, mask, v, y = _gcn_layer1(A, y0, encb0, encW1)
    x, y = _gcn_layer_masked(mask, v, y, encb1, x, decW0,
                             relu=False, residual=True)
    x, y = _gcn_layer_masked(mask, v, y, decb0, x, decW1,
                             relu=True, residual=False)
    x, _ = _gcn_layer_masked(mask, v, y, decb1, x, None,
                             relu=False, residual=True)

    n0, n1 = X0.shape[0], X1.shape[0]
    recon0, recon1, recon2, recon_loss = _heads(
        x[:n0], x[n0:n0 + n1], x[n0 + n1:],
        (fc2W0, fc2W1, fc2W2), (fc2b0, fc2b1, fc2b2), X0)

    emb_loss = _bce_loss(x, adj_full)
    return (recon0, recon1, recon2, emb_loss, recon_loss[0, 0])


# speedup vs baseline: 1.5717x; 1.0318x over previous
import functools

import jax
import jax.numpy as jnp
from jax.experimental import pallas as pl

N = 11616
NHID = 128
ROW_BLK = 352
N_BLKS = N // ROW_BLK
GAMMA = 2.0
CLIP_C = 16.11809565095832
_INTERPRET = False


def _encode_body(x0_ref, x1_ref, x2_ref, w0_ref, w1_ref, w2w_ref,
                 b0_ref, b1_ref, b2_ref, we_ref, y0_ref, y1_ref, y2_ref):
    we = we_ref[...]
    for x_ref, w_ref, b_ref, y_ref in (
            (x0_ref, w0_ref, b0_ref, y0_ref),
            (x1_ref, w1_ref, b1_ref, y1_ref),
            (x2_ref, w2w_ref, b2_ref, y2_ref)):
        t = (jnp.dot(x_ref[...], w_ref[...],
                     preferred_element_type=jnp.float32) + b_ref[...])
        y = jnp.dot(t, we, preferred_element_type=jnp.float32)
        y_ref[...] = y.astype(jnp.bfloat16)


def _encode_all(X0, X1, X2, fcW, fcb, encW0):
    return pl.pallas_call(
        _encode_body,
        out_shape=tuple(
            jax.ShapeDtypeStruct((x.shape[0], NHID), jnp.bfloat16)
            for x in (X0, X1, X2)),
        interpret=_INTERPRET,
    )(X0, X1, X2, fcW[0], fcW[1], fcW[2],
      fcb[0].reshape(1, NHID), fcb[1].reshape(1, NHID),
      fcb[2].reshape(1, NHID), encW0)


def _mm_bias_body(x_ref, w_ref, b_ref, o_ref):
    o_ref[...] = (
        jnp.dot(x_ref[...], w_ref[...], preferred_element_type=jnp.float32)
        + b_ref[...]
    )


def _mm_bias(x, w, b):
    m, _ = x.shape
    _, n = w.shape
    return pl.pallas_call(
        _mm_bias_body,
        out_shape=jax.ShapeDtypeStruct((m, n), jnp.float32),
        interpret=_INTERPRET,
    )(x, w, b.reshape(1, n))


def _gcn_extract_body(a_ref, y_ref, b_ref, w2_ref, o_ref, m_ref, v_ref,
                      y2_ref):
    a = a_ref[...]
    mb = a != 0.0
    m_ref[...] = mb.astype(jnp.int8)
    v = jnp.max(a, axis=1, keepdims=True)
    v_ref[...] = v
    s = jnp.dot(mb.astype(jnp.bfloat16), y_ref[...],
                preferred_element_type=jnp.float32)
    o = jnp.maximum(v * s + b_ref[...], 0.0)
    o_ref[...] = o
    y2 = jnp.dot(o, w2_ref[...], preferred_element_type=jnp.float32)
    y2_ref[...] = y2.astype(jnp.bfloat16)


def _gcn_layer1(A, y, b, w2):
    return pl.pallas_call(
        _gcn_extract_body,
        grid=(N_BLKS,),
        in_specs=[
            pl.BlockSpec((ROW_BLK, N), lambda i: (i, 0)),
            pl.BlockSpec((N, NHID), lambda i: (0, 0)),
            pl.BlockSpec((1, NHID), lambda i: (0, 0)),
            pl.BlockSpec((NHID, NHID), lambda i: (0, 0)),
        ],
        out_specs=(
            pl.BlockSpec((ROW_BLK, NHID), lambda i: (i, 0)),
            pl.BlockSpec((ROW_BLK, N), lambda i: (i, 0)),
            pl.BlockSpec((ROW_BLK, 1), lambda i: (i, 0)),
            pl.BlockSpec((ROW_BLK, NHID), lambda i: (i, 0)),
        ),
        out_shape=(
            jax.ShapeDtypeStruct((N, NHID), jnp.float32),
            jax.ShapeDtypeStruct((N, N), jnp.int8),
            jax.ShapeDtypeStruct((N, 1), jnp.float32),
            jax.ShapeDtypeStruct((N, NHID), jnp.bfloat16),
        ),
        interpret=_INTERPRET,
    )(A, y, b.reshape(1, NHID), w2)


def _gcn_mask_body(relu, residual, last, m_ref, v_ref, y_ref, b_ref,
                   x_ref, w2_ref, o_ref, y2_ref):
    s = jnp.dot(m_ref[...].astype(jnp.bfloat16), y_ref[...],
                preferred_element_type=jnp.float32)
    s = v_ref[...] * s + b_ref[...]
    if relu:
        s = jnp.maximum(s, 0.0)
    if residual:
        s = s + x_ref[...]
    o_ref[...] = s
    if not last:
        y2 = jnp.dot(s, w2_ref[...], preferred_element_type=jnp.float32)
        y2_ref[...] = y2.astype(jnp.bfloat16)


def _gcn_layer_masked(mask, v, y, b, x, w2, relu, residual):
    last = w2 is None
    body = functools.partial(_gcn_mask_body, relu, residual, last)
    if last:
        w2 = jnp.zeros((NHID, NHID), jnp.float32)
    out = pl.pallas_call(
        body,
        grid=(N_BLKS,),
        in_specs=[
            pl.BlockSpec((ROW_BLK, N), lambda i: (i, 0)),
            pl.BlockSpec((ROW_BLK, 1), lambda i: (i, 0)),
            pl.BlockSpec((N, NHID), lambda i: (0, 0)),
            pl.BlockSpec((1, NHID), lambda i: (0, 0)),
            pl.BlockSpec((ROW_BLK, NHID), lambda i: (i, 0)),
            pl.BlockSpec((NHID, NHID), lambda i: (0, 0)),
        ],
        out_specs=(
            pl.BlockSpec((ROW_BLK, NHID), lambda i: (i, 0)),
            pl.BlockSpec((ROW_BLK, NHID), lambda i: (i, 0)),
        ),
        out_shape=(
            jax.ShapeDtypeStruct((N, NHID), jnp.float32),
            jax.ShapeDtypeStruct((N, NHID), jnp.bfloat16),
        ),
        interpret=_INTERPRET,
    )(mask, v, y, b.reshape(1, NHID), x, w2)
    return out


def _bce_body(x_blk_ref, x_all_ref, adj_ref, o_ref):
    i = pl.program_id(0)
    z = jax.lax.dot_general(
        x_blk_ref[...], x_all_ref[...],
        (((1,), (1,)), ((), ())),
        preferred_element_type=jnp.float32,
    )
    a = adj_ref[...]
    sbit = jax.lax.shift_left(jax.lax.bitcast_convert_type(a, jnp.int32), 8)
    w = jax.lax.bitcast_convert_type(
        jax.lax.bitwise_xor(jax.lax.bitcast_convert_type(z, jnp.int32), sbit),
        jnp.float32).astype(jnp.bfloat16)
    sp = jnp.maximum(w, 0.0) + jnp.log1p(jnp.exp(-jnp.abs(w)))
    sp = jnp.minimum(sp, jnp.bfloat16(CLIP_C))
    part = jnp.sum(sp.astype(jnp.float32)).reshape(1, 1)

    @pl.when(i == 0)
    def _():
        o_ref[...] = jnp.zeros((1, 1), jnp.float32)

    o_ref[...] += part


def _bce_loss(x, adj_full):
    s = pl.pallas_call(
        _bce_body,
        grid=(N_BLKS,),
        in_specs=[
            pl.BlockSpec((ROW_BLK, NHID), lambda i: (i, 0)),
            pl.BlockSpec((N, NHID), lambda i: (0, 0)),
            pl.BlockSpec((ROW_BLK, N), lambda i: (i, 0)),
        ],
        out_specs=pl.BlockSpec((1, 1), lambda i: (0, 0)),
        out_shape=jax.ShapeDtypeStruct((1, 1), jnp.float32),
        interpret=_INTERPRET,
    )(x, x, adj_full)
    return s[0, 0] / (float(N) * float(N))


def _heads_body(e0_ref, e1_ref, e2_ref, w0_ref, w1_ref, w2_ref,
                b0_ref, b1_ref, b2_ref, f_ref,
                o0_ref, o1_ref, o2_ref, l_ref):
    r = (jnp.dot(e0_ref[...], w0_ref[...],
                 preferred_element_type=jnp.float32) + b0_ref[...])
    o0_ref[...] = r
    f = f_ref[...]
    rn = jnp.maximum(jnp.sqrt(jnp.sum(r * r, axis=-1, keepdims=True)), 1e-12)
    fn = jnp.maximum(jnp.sqrt(jnp.sum(f * f, axis=-1, keepdims=True)), 1e-12)
    cs = jnp.sum((r / rn) * (f / fn), axis=-1)
    l_ref[...] = jnp.mean((1.0 - cs) ** GAMMA).reshape(1, 1)
    o1_ref[...] = (jnp.dot(e1_ref[...], w1_ref[...],
                           preferred_element_type=jnp.float32) + b1_ref[...])
    o2_ref[...] = (jnp.dot(e2_ref[...], w2_ref[...],
                           preferred_element_type=jnp.float32) + b2_ref[...])


def _heads(e0, e1, e2, fc2W, fc2b, X0):
    fo = X0.shape[1]
    return pl.pallas_call(
        _heads_body,
        out_shape=(
            jax.ShapeDtypeStruct((e0.shape[0], fc2W[0].shape[1]), jnp.float32),
            jax.ShapeDtypeStruct((e1.shape[0], fc2W[1].shape[1]), jnp.float32),
            jax.ShapeDtypeStruct((e2.shape[0], fc2W[2].shape[1]), jnp.float32),
            jax.ShapeDtypeStruct((1, 1), jnp.float32),
        ),
        interpret=_INTERPRET,
    )(e0, e1, e2, fc2W[0], fc2W[1], fc2W[2],
      fc2b[0].reshape(1, fc2W[0].shape[1]),
      fc2b[1].reshape(1, fc2W[1].shape[1]),
      fc2b[2].reshape(1, fc2W[2].shape[1]), X0)


def kernel(A, adj_full, X0, X1, X2, fcW0, fcb0, fcW1, fcb1, fcW2, fcb2,
           encW0, encb0, encW1, encb1, decW0, decb0, decW1, decb1,
           fc2W0, fc2b0, fc2W1, fc2b1, fc2W2, fc2b2):
    y0 = jnp.concatenate(
        _encode_all(X0, X1, X2, (fcW0, fcW1, fcW2), (fcb0, fcb1, fcb2),
                    encW0),
        axis=0)

    x, mask, v, y = _gcn_layer1(A, y0, encb0, encW1)
    x, y = _gcn_layer_masked(mask, v, y, encb1, x, decW0,
                             relu=False, residual=True)
    x, y = _gcn_layer_masked(mask, v, y, decb0, x, decW1,
                             relu=True, residual=False)
    x, _ = _gcn_layer_masked(mask, v, y, decb1, x, None,
                             relu=False, residual=True)

    n0, n1 = X0.shape[0], X1.shape[0]
    recon0, recon1, recon2, recon_loss = _heads(
        x[:n0], x[n0:n0 + n1], x[n0 + n1:],
        (fc2W0, fc2W1, fc2W2), (fc2b0, fc2b1, fc2b2), X0)

    emb_loss = _bce_loss(x, adj_full)
    return (recon0, recon1, recon2, emb_loss, recon_loss[0, 0])
